# baseline (speedup 1.0000x reference)
import functools

import jax
import jax.numpy as jnp
from jax import lax
from jax.experimental import pallas as pl
from jax.experimental.pallas import tpu as pltpu
from jax.experimental.pallas import tpu_sc as plsc

_NC = 2
_NS = 16
_NW = _NC * _NS


def _sc_gather_body(bpw, uid_h, iid_h, umf_h, imf_h, umlp_h, imlp_h, ub_h, ib_h,
                    umf_o, imf_o, umlp_o, imlp_o, ub_o, ib_o,
                    uid_v, iid_v, umf_v, imf_v, umlp_v, imlp_v, ub_v, ib_v, sem):
    wid = lax.axis_index("s") * _NC + lax.axis_index("c")
    base = wid * bpw
    pltpu.sync_copy(uid_h.at[pl.ds(base, bpw)], uid_v)
    pltpu.sync_copy(iid_h.at[pl.ds(base, bpw)], iid_v)
    c1 = pltpu.async_copy(umf_h.at[uid_v], umf_v, sem)
    c2 = pltpu.async_copy(imf_h.at[iid_v], imf_v, sem)
    c3 = pltpu.async_copy(umlp_h.at[uid_v], umlp_v, sem)
    c4 = pltpu.async_copy(imlp_h.at[iid_v], imlp_v, sem)
    c5 = pltpu.async_copy(ub_h.at[uid_v], ub_v, sem)
    c6 = pltpu.async_copy(ib_h.at[iid_v], ib_v, sem)
    c1.wait()
    c2.wait()
    c3.wait()
    c4.wait()
    c5.wait()
    c6.wait()
    pltpu.sync_copy(umf_v, umf_o.at[pl.ds(base, bpw)])
    pltpu.sync_copy(imf_v, imf_o.at[pl.ds(base, bpw)])
    pltpu.sync_copy(umlp_v, umlp_o.at[pl.ds(base, bpw)])
    pltpu.sync_copy(imlp_v, imlp_o.at[pl.ds(base, bpw)])
    pltpu.sync_copy(ub_v, ub_o.at[pl.ds(base, bpw)])
    pltpu.sync_copy(ib_v, ib_o.at[pl.ds(base, bpw)])


def _sc_gather(user_ids, item_ids, umf, imf, umlp, imlp, ub, ib):
    b = user_ids.shape[0]
    bpw = b // _NW
    d_mf = umf.shape[1]
    d_mlp = umlp.shape[1]
    f32 = jnp.float32
    mesh = plsc.VectorSubcoreMesh(core_axis_name="c", subcore_axis_name="s")
    k = pl.kernel(
        functools.partial(_sc_gather_body, bpw),
        out_type=[
            jax.ShapeDtypeStruct((b, d_mf), f32),
            jax.ShapeDtypeStruct((b, d_mf), f32),
            jax.ShapeDtypeStruct((b, d_mlp), f32),
            jax.ShapeDtypeStruct((b, d_mlp), f32),
            jax.ShapeDtypeStruct((b,), f32),
            jax.ShapeDtypeStruct((b,), f32),
        ],
        mesh=mesh,
        compiler_params=pltpu.CompilerParams(use_tc_tiling_on_sc=False),
        scratch_types=[
            pltpu.VMEM((bpw,), jnp.int32),
            pltpu.VMEM((bpw,), jnp.int32),
            pltpu.VMEM((bpw, d_mf), f32),
            pltpu.VMEM((bpw, d_mf), f32),
            pltpu.VMEM((bpw, d_mlp), f32),
            pltpu.VMEM((bpw, d_mlp), f32),
            pltpu.VMEM((bpw,), f32),
            pltpu.VMEM((bpw,), f32),
            pltpu.SemaphoreType.DMA,
        ],
    )
    return k(user_ids, item_ids, umf, imf, umlp, imlp,
             ub.reshape(-1), ib.reshape(-1))


def _tc_mlp_body(umf, imf, umlp, imlp, ub, ib, w1, b1, w2, b2, w3, b3, wo, bo, out):
    d_mlp = umlp.shape[1]
    d_mf = umf.shape[1]
    dn = (((1,), (1,)), ((), ()))
    h = lax.dot_general(umlp[...], w1[:, :d_mlp], dn,
                        preferred_element_type=jnp.float32)
    h += lax.dot_general(imlp[...], w1[:, d_mlp:], dn,
                         preferred_element_type=jnp.float32)
    h = jnp.maximum(h + b1[...], 0.0)
    h = jnp.maximum(
        lax.dot_general(h, w2[...], dn, preferred_element_type=jnp.float32)
        + b2[...], 0.0)
    h = jnp.maximum(
        lax.dot_general(h, w3[...], dn, preferred_element_type=jnp.float32)
        + b3[...], 0.0)
    gmf = umf[...] * imf[...]
    o = lax.dot_general(gmf, wo[:, :d_mf], dn, preferred_element_type=jnp.float32)
    o += lax.dot_general(h, wo[:, d_mf:], dn, preferred_element_type=jnp.float32)
    out[...] = o + bo[...] + ub[...] + ib[...]


def _tc_mlp(umf_r, imf_r, umlp_r, imlp_r, ub_r, ib_r, W1, b1, W2, b2, W3, b3,
            W_out, b_out):
    b = umf_r.shape[0]
    bb = 2048
    grid = (b // bb,)
    d_mf = umf_r.shape[1]
    d_mlp = umlp_r.shape[1]

    def batch_spec(w):
        return pl.BlockSpec((bb, w), lambda i: (i, 0))

    def full_spec(a):
        return pl.BlockSpec(a.shape, lambda i: (0,) * a.ndim)

    return pl.pallas_call(
        _tc_mlp_body,
        grid=grid,
        in_specs=[
            batch_spec(d_mf), batch_spec(d_mf),
            batch_spec(d_mlp), batch_spec(d_mlp),
            batch_spec(1), batch_spec(1),
            full_spec(W1), full_spec(b1), full_spec(W2), full_spec(b2),
            full_spec(W3), full_spec(b3), full_spec(W_out), full_spec(b_out),
        ],
        out_specs=batch_spec(1),
        out_shape=jax.ShapeDtypeStruct((b, 1), jnp.float32),
    )(umf_r, imf_r, umlp_r, imlp_r, ub_r, ib_r, W1, b1, W2, b2, W3, b3,
      W_out, b_out)


def kernel(user_ids, item_ids, user_mf_emb, item_mf_emb, user_mlp_emb,
           item_mlp_emb, user_bias_emb, item_bias_emb, W1, b1, W2, b2, W3, b3,
           W_out, b_out):
    umf_r, imf_r, umlp_r, imlp_r, ub_r, ib_r = _sc_gather(
        user_ids, item_ids, user_mf_emb, item_mf_emb, user_mlp_emb,
        item_mlp_emb, user_bias_emb, item_bias_emb)
    ub_r = ub_r.reshape(-1, 1)
    ib_r = ib_r.reshape(-1, 1)
    return _tc_mlp(umf_r, imf_r, umlp_r, imlp_r, ub_r, ib_r,
                   W1, b1.reshape(1, -1), W2, b2.reshape(1, -1),
                   W3, b3.reshape(1, -1), W_out, b_out.reshape(1, -1))
